# baseline (speedup 1.0000x reference)
import jax
import jax.numpy as jnp
from jax.experimental import pallas as pl
from jax.experimental.pallas import tpu as pltpu

N = 4096
NFEAT = 512
NHID = 256
NCLASS = 64

BX = 2048
BM = 512
BO = 1024
KB = 1024
P1 = N // BX
P2 = P1 + N // BM
GRID = P2 + N // BO

_SHADOW = {2: ((0, 0),), 3: ((1, 0),),
           4: ((0, 1), (1, 1)), 5: ((2, 0), (0, 2)),
           6: ((2, 1), (1, 2)), 7: ((3, 0), (2, 2))}
_REMAIN = {0: (3,), 1: (3,), 2: (3,), 3: (1, 2, 3)}


def _gcn_kernel(x_ref, adj_ref, w1_ref, b1_ref, w2_ref, b2_ref,
                x1_ref, out_ref, s1_ref, q_ref, s2_ref, h2_ref):
    i = pl.program_id(0)

    def _partial(m, c, init):
        qb = q_ref[m * BO:(m + 1) * BO, c * KB:(c + 1) * KB].astype(
            jnp.bfloat16
        )
        d = jnp.dot(qb, s2_ref[c * KB:(c + 1) * KB, :],
                    preferred_element_type=jnp.float32)
        if init:
            h2_ref[m * BO:(m + 1) * BO, :] = d
        else:
            h2_ref[m * BO:(m + 1) * BO, :] = h2_ref[m * BO:(m + 1) * BO, :] + d

    @pl.when(i < P1)
    def _phase0():
        xb = x_ref[...].astype(jnp.bfloat16)
        wb = w1_ref[...].astype(jnp.bfloat16)
        s1_ref[pl.ds(i * BX, BX), :] = jnp.dot(
            xb, wb, preferred_element_type=jnp.float32
        ).astype(jnp.bfloat16)

    @pl.when(jnp.logical_and(i >= P1, i < P2))
    def _phase1():
        r = i - P1
        adj = adj_ref[...]
        q_ref[pl.ds(r * BM, BM), :] = jnp.round(adj * 255.0 - 127.5).astype(
            jnp.int8
        )
        h = jnp.dot(
            adj.astype(jnp.bfloat16), s1_ref[...],
            preferred_element_type=jnp.float32,
        )
        x1 = jnp.maximum(h + b1_ref[...], 0.0)
        x1_ref[...] = x1
        s2_ref[pl.ds(r * BM, BM), :] = jnp.dot(
            x1.astype(jnp.bfloat16), w2_ref[...],
            preferred_element_type=jnp.float32,
        ).astype(jnp.bfloat16)

        for rr, pairs in _SHADOW.items():
            @pl.when(r == rr)
            def _shadow(pairs=pairs):
                for m, c in pairs:
                    _partial(m, c, init=(c == 0))

    @pl.when(i >= P2)
    def _phase2():
        m = i - P2
        s2 = s2_ref[...]
        cs = jnp.sum(s2.astype(jnp.float32), axis=0, keepdims=True)
        for mm, cbs in _REMAIN.items():
            @pl.when(m == mm)
            def _rem(mm=mm, cbs=cbs):
                for c in cbs:
                    _partial(mm, c, init=False)
        acc = h2_ref[pl.ds(m * BO, BO), :]
        h2 = acc * (1.0 / 255.0) + 0.5 * cs + b2_ref[...]
        x2 = jnp.maximum(h2, 0.0)
        mx = jnp.max(x2, axis=1, keepdims=True)
        lse = jnp.log(jnp.sum(jnp.exp(x2 - mx), axis=1, keepdims=True))
        out_ref[...] = x2 - mx - lse


def kernel(x, adj, gc1_W, gc1_b, gc2_W, gc2_b):
    b1 = gc1_b.reshape(1, NHID)
    b2 = gc2_b.reshape(1, NCLASS)
    w2 = gc2_W.astype(jnp.bfloat16)

    x1, out = pl.pallas_call(
        _gcn_kernel,
        grid=(GRID,),
        in_specs=[
            pl.BlockSpec((BX, NFEAT), lambda i: (jnp.minimum(i, P1 - 1), 0)),
            pl.BlockSpec(
                (BM, N), lambda i: (jnp.clip(i - P1, 0, N // BM - 1), 0)
            ),
            pl.BlockSpec((NFEAT, NHID), lambda i: (0, 0)),
            pl.BlockSpec((1, NHID), lambda i: (0, 0)),
            pl.BlockSpec((NHID, NCLASS), lambda i: (0, 0)),
            pl.BlockSpec((1, NCLASS), lambda i: (0, 0)),
        ],
        out_specs=[
            pl.BlockSpec(
                (BM, NHID), lambda i: (jnp.clip(i - P1, 0, N // BM - 1), 0)
            ),
            pl.BlockSpec(
                (BO, NCLASS), lambda i: (jnp.clip(i - P2, 0, N // BO - 1), 0)
            ),
        ],
        out_shape=[
            jax.ShapeDtypeStruct((N, NHID), jnp.float32),
            jax.ShapeDtypeStruct((N, NCLASS), jnp.float32),
        ],
        scratch_shapes=[
            pltpu.VMEM((N, NHID), jnp.bfloat16),
            pltpu.VMEM((N, N), jnp.int8),
            pltpu.VMEM((N, NCLASS), jnp.bfloat16),
            pltpu.VMEM((N, NCLASS), jnp.float32),
        ],
    )(x, adj, gc1_W, b1, w2, b2)

    return (out, x1)


# speedup vs baseline: 1.0525x; 1.0525x over previous
import jax
import jax.numpy as jnp
from jax.experimental import pallas as pl
from jax.experimental.pallas import tpu as pltpu

N = 4096
NFEAT = 512
NHID = 256
NCLASS = 64
NH = N // 2

BX = 2048
BM = 512
BO = 1024
P1 = N // BX
P2 = P1 + N // BM
GRID = P2 + N // BO


def _gcn_kernel(x_ref, adj_ref, w1_ref, b1_ref, w2_ref, b2_ref,
                x1_ref, out_ref, s1_ref, q_ref, s2_ref, h2_ref):
    i = pl.program_id(0)

    @pl.when(i < P1)
    def _phase0():
        xb = x_ref[...].astype(jnp.bfloat16)
        wb = w1_ref[...].astype(jnp.bfloat16)
        s1_ref[pl.ds(i * BX, BX), :] = jnp.dot(
            xb, wb, preferred_element_type=jnp.float32
        ).astype(jnp.bfloat16)

    @pl.when(jnp.logical_and(i >= P1, i < P2))
    def _phase1():
        r = i - P1
        adj = adj_ref[...]
        q_ref[pl.ds(r * BM, BM), :] = jnp.round(adj * 255.0 - 127.5).astype(
            jnp.int8
        )
        h = jnp.dot(
            adj.astype(jnp.bfloat16), s1_ref[...],
            preferred_element_type=jnp.float32,
        )
        x1 = jnp.maximum(h + b1_ref[...], 0.0)
        x1_ref[...] = x1
        s2_ref[pl.ds(r * BM, BM), :] = jnp.dot(
            x1.astype(jnp.bfloat16), w2_ref[...],
            preferred_element_type=jnp.float32,
        ).astype(jnp.bfloat16)

        @pl.when(r >= 4)
        def _half_h2():
            m = r - 4
            qb = q_ref[pl.ds(m * BO, BO), :NH].astype(jnp.bfloat16)
            h2_ref[pl.ds(m * BO, BO), :] = jnp.dot(
                qb, s2_ref[:NH, :], preferred_element_type=jnp.float32
            )

    @pl.when(i >= P2)
    def _phase2():
        m = i - P2
        s2 = s2_ref[...]
        cs = jnp.sum(s2.astype(jnp.float32), axis=0, keepdims=True)
        qb = q_ref[pl.ds(m * BO, BO), NH:].astype(jnp.bfloat16)
        acc = h2_ref[pl.ds(m * BO, BO), :] + jnp.dot(
            qb, s2_ref[NH:, :], preferred_element_type=jnp.float32
        )
        h2 = acc * (1.0 / 255.0) + 0.5 * cs + b2_ref[...]
        x2 = jnp.maximum(h2, 0.0)
        m_ = jnp.max(x2, axis=1, keepdims=True)
        lse = jnp.log(jnp.sum(jnp.exp(x2 - m_), axis=1, keepdims=True))
        out_ref[...] = x2 - m_ - lse


def kernel(x, adj, gc1_W, gc1_b, gc2_W, gc2_b):
    b1 = gc1_b.reshape(1, NHID)
    b2 = gc2_b.reshape(1, NCLASS)
    w2 = gc2_W.astype(jnp.bfloat16)

    x1, out = pl.pallas_call(
        _gcn_kernel,
        grid=(GRID,),
        in_specs=[
            pl.BlockSpec((BX, NFEAT), lambda i: (jnp.minimum(i, P1 - 1), 0)),
            pl.BlockSpec(
                (BM, N), lambda i: (jnp.clip(i - P1, 0, N // BM - 1), 0)
            ),
            pl.BlockSpec((NFEAT, NHID), lambda i: (0, 0)),
            pl.BlockSpec((1, NHID), lambda i: (0, 0)),
            pl.BlockSpec((NHID, NCLASS), lambda i: (0, 0)),
            pl.BlockSpec((1, NCLASS), lambda i: (0, 0)),
        ],
        out_specs=[
            pl.BlockSpec(
                (BM, NHID), lambda i: (jnp.clip(i - P1, 0, N // BM - 1), 0)
            ),
            pl.BlockSpec(
                (BO, NCLASS), lambda i: (jnp.clip(i - P2, 0, N // BO - 1), 0)
            ),
        ],
        out_shape=[
            jax.ShapeDtypeStruct((N, NHID), jnp.float32),
            jax.ShapeDtypeStruct((N, NCLASS), jnp.float32),
        ],
        scratch_shapes=[
            pltpu.VMEM((N, NHID), jnp.bfloat16),
            pltpu.VMEM((N, N), jnp.int8),
            pltpu.VMEM((N, NCLASS), jnp.bfloat16),
            pltpu.VMEM((N, NCLASS), jnp.float32),
        ],
    )(x, adj, gc1_W, b1, w2, b2)

    return (out, x1)
